# Initial kernel scaffold; baseline (speedup 1.0000x reference)
import jax
import jax.numpy as jnp
from jax.experimental import pallas as pl


def kernel(x, emb_region, emb_gene, emb_strand, emb_exon, W_proj, b_proj):
    raise NotImplementedError("write your pallas kernel here")


# speedup vs baseline: 9.4971x; 9.4971x over previous
import functools

import jax
import jax.numpy as jnp
from jax import lax
from jax.experimental import pallas as pl
from jax.experimental.pallas import tpu as pltpu
from jax.experimental.pallas import tpu_sc as plsc

NUM_EMB = 100000
D = 128
B = 4096
L = 200
T = B * L


_ROWS_BLK = 2000
_PROJ_STEPS = NUM_EMB // _ROWS_BLK


def _proj_body(e1, e2, e3, e4, w1, w2, w3, w4, p1, p2, p3, p4):
    p1[...] = jnp.dot(e1[...], w1[...], preferred_element_type=jnp.float32)
    p2[...] = jnp.dot(e2[...], w2[...], preferred_element_type=jnp.float32)
    p3[...] = jnp.dot(e3[...], w3[...], preferred_element_type=jnp.float32)
    p4[...] = jnp.dot(e4[...], w4[...], preferred_element_type=jnp.float32)


def _project_tables(e1, e2, e3, e4, w1, w2, w3, w4):
    blk = pl.BlockSpec((_ROWS_BLK, D), lambda i: (i, 0))
    wblk = pl.BlockSpec((D, D), lambda i: (0, 0))
    shp = jax.ShapeDtypeStruct((NUM_EMB, D), jnp.float32)
    return pl.pallas_call(
        _proj_body,
        grid=(_PROJ_STEPS,),
        in_specs=[blk, blk, blk, blk, wblk, wblk, wblk, wblk],
        out_specs=[blk, blk, blk, blk],
        out_shape=[shp, shp, shp, shp],
    )(e1, e2, e3, e4, w1, w2, w3, w4)


_TOK_BLK = 16384


def _base_body(nuc, w0, bias, out):
    out[...] = (
        jnp.dot(nuc[...], w0[...], preferred_element_type=jnp.float32)
        + bias[...]
    )


def _token_base(nuc, w0, bias):
    return pl.pallas_call(
        _base_body,
        grid=(T // _TOK_BLK,),
        in_specs=[
            pl.BlockSpec((_TOK_BLK, 4), lambda i: (i, 0)),
            pl.BlockSpec((4, D), lambda i: (0, 0)),
            pl.BlockSpec((1, D), lambda i: (0, 0)),
        ],
        out_specs=pl.BlockSpec((_TOK_BLK, D), lambda i: (i, 0)),
        out_shape=jax.ShapeDtypeStruct((T, D), jnp.float32),
    )(nuc, w0, bias)


_CHUNK = 512
_GRP = 128
_K = _CHUNK // _GRP


def _gather_sum(p1, p2, p3, p4, idx3d, base):
    info = plsc.get_sparse_core_info()
    nc, ns = info.num_cores, info.num_subcores
    nw = nc * ns
    tpw = T // nw
    nch = tpw // _CHUNK
    mesh = plsc.VectorSubcoreMesh(core_axis_name="c", subcore_axis_name="s")

    nouter = tpw // (2 * _CHUNK)

    @functools.partial(
        pl.kernel,
        out_type=jax.ShapeDtypeStruct((T, D), jnp.float32),
        mesh=mesh,
        scratch_types=[
            pltpu.VMEM((4, 2 * _K, _GRP), jnp.int32),
            pltpu.VMEM((_CHUNK, D), jnp.float32),
            pltpu.SemaphoreType.DMA,
        ],
    )
    def k(p1h, p2h, p3h, p4h, idxh, baseh, outh, idx_v, acc_v, sem):
        wid = lax.axis_index("s") * nc + lax.axis_index("c")
        w_tok = wid * tpw
        tables = (p1h, p2h, p3h, p4h)

        def outer(g, carry):
            row = pl.multiple_of(w_tok // _GRP + g * (2 * _K), 8)
            for i in range(4):
                pltpu.sync_copy(idxh.at[i, pl.ds(row, 2 * _K)], idx_v.at[i])
            for s in range(2):
                tok = pl.multiple_of(w_tok + (2 * g + s) * _CHUNK, _CHUNK)
                pltpu.sync_copy(baseh.at[pl.ds(tok, _CHUNK)], acc_v)
                cps = []
                for i in range(4):
                    for j in range(_K):
                        cps.append(pltpu.async_copy(
                            tables[i].at[idx_v.at[i, s * _K + j]],
                            acc_v.at[pl.ds(j * _GRP, _GRP)],
                            sem, add=True))
                for cp in cps:
                    cp.wait()
                pltpu.sync_copy(acc_v, outh.at[pl.ds(tok, _CHUNK)])
            return carry

        lax.fori_loop(0, nouter, outer, 0)

    return k(p1, p2, p3, p4, idx3d, base)


def kernel(x, emb_region, emb_gene, emb_strand, emb_exon, W_proj, b_proj):
    xf = x.reshape(T, 9)
    nuc = xf[:, :4]
    idx3d = xf[:, 4:8].astype(jnp.int32).T.reshape(4, T // _GRP, _GRP)
    w0 = W_proj[:4]
    w1 = W_proj[4:132]
    w2 = W_proj[132:260]
    w3 = W_proj[260:388]
    w4 = W_proj[388:516]
    p1, p2, p3, p4 = _project_tables(
        emb_region, emb_gene, emb_strand, emb_exon, w1, w2, w3, w4)
    base = _token_base(nuc, w0, b_proj.reshape(1, D))
    out = _gather_sum(p1, p2, p3, p4, idx3d, base)
    return out.reshape(B, L, D)


# speedup vs baseline: 15.4884x; 1.6309x over previous
import functools

import jax
import jax.numpy as jnp
from jax import lax
from jax.experimental import pallas as pl
from jax.experimental.pallas import tpu as pltpu
from jax.experimental.pallas import tpu_sc as plsc

NUM_EMB = 100000
D = 128
B = 4096
L = 200
T = B * L
NF = 9


_ROWS_BLK = 2000
_PROJ_STEPS = NUM_EMB // _ROWS_BLK


def _proj_body(e1, e2, e3, e4, w1, w2, w3, w4, p1, p2, p3, p4):
    p1[...] = jnp.dot(e1[...], w1[...], preferred_element_type=jnp.float32)
    p2[...] = jnp.dot(e2[...], w2[...], preferred_element_type=jnp.float32)
    p3[...] = jnp.dot(e3[...], w3[...], preferred_element_type=jnp.float32)
    p4[...] = jnp.dot(e4[...], w4[...], preferred_element_type=jnp.float32)


def _project_tables(e1, e2, e3, e4, w1, w2, w3, w4):
    blk = pl.BlockSpec((_ROWS_BLK, D), lambda i: (i, 0))
    wblk = pl.BlockSpec((D, D), lambda i: (0, 0))
    shp = jax.ShapeDtypeStruct((NUM_EMB, D), jnp.float32)
    return pl.pallas_call(
        _proj_body,
        grid=(_PROJ_STEPS,),
        in_specs=[blk, blk, blk, blk, wblk, wblk, wblk, wblk],
        out_specs=[blk, blk, blk, blk],
        out_shape=[shp, shp, shp, shp],
    )(e1, e2, e3, e4, w1, w2, w3, w4)


_C = 256
_GRP = 128
_K = _C // _GRP
_LANES = 16


def _gather_sum(p1, p2, p3, p4, x1d, w0b):
    info = plsc.get_sparse_core_info()
    nc, ns = info.num_cores, info.num_subcores
    nw = nc * ns
    tpw = T // nw
    n2 = tpw // (2 * _C)
    mesh = plsc.VectorSubcoreMesh(core_axis_name="c", subcore_axis_name="s")

    @functools.partial(
        pl.kernel,
        out_type=jax.ShapeDtypeStruct((T, D), jnp.float32),
        mesh=mesh,
        scratch_types=[
            pltpu.VMEM((_C * NF,), jnp.float32),
            pltpu.VMEM((_C * NF,), jnp.float32),
            pltpu.VMEM((4, _K, _GRP), jnp.int32),
            pltpu.VMEM((4, _K, _GRP), jnp.int32),
            pltpu.VMEM((_C, D), jnp.float32),
            pltpu.VMEM((_C, D), jnp.float32),
            pltpu.VMEM((5, D), jnp.float32),
            pltpu.SemaphoreType.DMA,
        ],
        compiler_params=pltpu.CompilerParams(needs_layout_passes=False),
    )
    def k(p1h, p2h, p3h, p4h, xh, wh, outh,
          xb0, xb1, idx0, idx1, acc0, acc1, w0v, sem):
        wid = lax.axis_index("s") * nc + lax.axis_index("c")
        w_tok = wid * tpw
        tables = (p1h, p2h, p3h, p4h)
        xbs, idxs, accs = (xb0, xb1), (idx0, idx1), (acc0, acc1)

        pltpu.sync_copy(wh, w0v)
        iota9 = lax.iota(jnp.int32, _LANES) * NF

        def prepare(s, tok):
            xb, idx_v, acc = xbs[s], idxs[s], accs[s]
            pltpu.sync_copy(xh.at[pl.ds(tok * NF, _C * NF)], xb)
            for kk in range(4):
                for m in range(_C // _LANES):
                    vals = plsc.load_gather(
                        xb, [iota9 + (m * _LANES * NF + 4 + kk)])
                    idx_v[kk, m * _LANES // _GRP,
                          pl.ds((m * _LANES) % _GRP, _LANES)] = (
                        vals.astype(jnp.int32))
            wvec = [[w0v[j, pl.ds(cc * _LANES, _LANES)] for j in range(4)]
                    for cc in range(D // _LANES)]
            bvec = [w0v[4, pl.ds(cc * _LANES, _LANES)]
                    for cc in range(D // _LANES)]

            def tokbody(t, carry):
                n = [plsc.load_gather(
                        xb, [jnp.full((_LANES,), t * NF + j, jnp.int32)])
                     for j in range(4)]
                for cc in range(D // _LANES):
                    v = bvec[cc]
                    for j in range(4):
                        v = v + n[j] * wvec[cc][j]
                    acc[t, pl.ds(cc * _LANES, _LANES)] = v
                return carry

            lax.fori_loop(0, _C, tokbody, 0)

        def fire(s):
            idx_v, acc = idxs[s], accs[s]
            cps = []
            for i in range(4):
                for j in range(_K):
                    cps.append(pltpu.async_copy(
                        tables[i].at[idx_v.at[i, j]],
                        acc.at[pl.ds(j * _GRP, _GRP)],
                        sem, add=True))
            return cps

        def body(g2, carry):
            tok_a = pl.multiple_of(w_tok + g2 * (2 * _C), _C)
            tok_b = pl.multiple_of(tok_a + _C, _C)
            d_a = fire(0)
            prepare(1, tok_b)
            for cp in d_a:
                cp.wait()
            d_b = fire(1)
            pltpu.sync_copy(accs[0], outh.at[pl.ds(tok_a, _C)])

            @pl.when(g2 < n2 - 1)
            def _():
                prepare(0, pl.multiple_of(tok_a + 2 * _C, _C))

            for cp in d_b:
                cp.wait()
            pltpu.sync_copy(accs[1], outh.at[pl.ds(tok_b, _C)])
            return carry

        prepare(0, w_tok)
        lax.fori_loop(0, n2, body, 0)

    return k(p1, p2, p3, p4, x1d, w0b)


def kernel(x, emb_region, emb_gene, emb_strand, emb_exon, W_proj, b_proj):
    x1d = x.reshape(T * NF)
    w1 = W_proj[4:132]
    w2 = W_proj[132:260]
    w3 = W_proj[260:388]
    w4 = W_proj[388:516]
    w0b = jnp.concatenate([W_proj[:4], b_proj.reshape(1, D)], axis=0)
    p1, p2, p3, p4 = _project_tables(
        emb_region, emb_gene, emb_strand, emb_exon, w1, w2, w3, w4)
    out = _gather_sum(p1, p2, p3, p4, x1d, w0b)
    return out.reshape(B, L, D)


# speedup vs baseline: 20.9121x; 1.3502x over previous
import functools

import jax
import jax.numpy as jnp
from jax import lax
from jax.experimental import pallas as pl
from jax.experimental.pallas import tpu as pltpu
from jax.experimental.pallas import tpu_sc as plsc

NUM_EMB = 100000
D = 128
NB = 4096
NL = 200
T = NB * NL
NF = 9
LN = 16


_ROWS_BLK = 2000
_PROJ_STEPS = NUM_EMB // _ROWS_BLK


def _proj_body(e1, e2, e3, e4, w1, w2, w3, w4, p1, p2, p3, p4):
    p1[...] = jnp.dot(e1[...], w1[...], preferred_element_type=jnp.float32)
    p2[...] = jnp.dot(e2[...], w2[...], preferred_element_type=jnp.float32)
    p3[...] = jnp.dot(e3[...], w3[...], preferred_element_type=jnp.float32)
    p4[...] = jnp.dot(e4[...], w4[...], preferred_element_type=jnp.float32)


def _project_tables(e1, e2, e3, e4, w1, w2, w3, w4):
    blk = pl.BlockSpec((_ROWS_BLK, D), lambda i: (i, 0))
    wblk = pl.BlockSpec((D, D), lambda i: (0, 0))
    shp = jax.ShapeDtypeStruct((NUM_EMB, D), jnp.float32)
    return pl.pallas_call(
        _proj_body,
        grid=(_PROJ_STEPS,),
        in_specs=[blk, blk, blk, blk, wblk, wblk, wblk, wblk],
        out_specs=[blk, blk, blk, blk],
        out_shape=[shp, shp, shp, shp],
    )(e1, e2, e3, e4, w1, w2, w3, w4)


_C = 128


def _gather_sum(p1, p2, p3, p4, xt1, w0b):
    info = plsc.get_sparse_core_info()
    nc, ns = info.num_cores, info.num_subcores
    nw = nc * ns
    assert NB == nw * _C
    n2 = NL // 2
    mesh = plsc.VectorSubcoreMesh(core_axis_name="c", subcore_axis_name="s")

    @functools.partial(
        pl.kernel,
        out_type=jax.ShapeDtypeStruct((T, D), jnp.float32),
        mesh=mesh,
        scratch_types=[
            pltpu.VMEM((8 * _C,), jnp.float32),
            pltpu.VMEM((8 * _C,), jnp.float32),
            pltpu.VMEM((5, _C), jnp.int32),
            pltpu.VMEM((5, _C), jnp.int32),
            pltpu.VMEM((_C, D), jnp.float32),
            pltpu.VMEM((_C, D), jnp.float32),
            pltpu.VMEM((5, D), jnp.float32),
            pltpu.SemaphoreType.DMA,
            pltpu.SemaphoreType.DMA,
            pltpu.SemaphoreType.DMA,
            pltpu.SemaphoreType.DMA,
        ],
        compiler_params=pltpu.CompilerParams(needs_layout_passes=False),
    )
    def k(p1h, p2h, p3h, p4h, xh, wh, outh,
          xb0, xb1, idx0, idx1, acc0, acc1, w0v,
          xsem, gsem, osem0, osem1):
        wid = lax.axis_index("s") * nc + lax.axis_index("c")
        b0 = wid * _C
        tables = (p1h, p2h, p3h, p4h)
        xbs, idxs, accs = (xb0, xb1), (idx0, idx1), (acc0, acc1)
        osems = (osem0, osem1)

        pltpu.sync_copy(wh, w0v)
        iota200 = lax.iota(jnp.int32, LN) * NL
        wvec = [[w0v[j, pl.ds(cc * LN, LN)] for j in range(4)]
                for cc in range(D // LN)]
        bvec = [w0v[4, pl.ds(cc * LN, LN)] for cc in range(D // LN)]

        def scatter_desc(s):
            return pltpu.make_async_copy(
                accs[s], outh.at[idxs[s].at[4]], osems[s])

        def prepare(s, l, drain):
            xb, idx_v, acc = xbs[s], idxs[s], accs[s]
            dx = []
            for j in range(8):
                off = pl.multiple_of((j * NL + l) * NB + b0, _C)
                dx.append(pltpu.async_copy(
                    xh.at[pl.ds(off, _C)], xb.at[pl.ds(j * _C, _C)], xsem))
            for d in dx:
                d.wait()
            for kk in range(4):
                for m in range(_C // LN):
                    v = xb[pl.ds((4 + kk) * _C + m * LN, LN)]
                    idx_v[kk, pl.ds(m * LN, LN)] = v.astype(jnp.int32)
            for m in range(_C // LN):
                idx_v[4, pl.ds(m * LN, LN)] = (
                    iota200 + ((b0 + m * LN) * NL + l))
            if drain:
                scatter_desc(s).wait()
            def tokbody(t, carry):
                n = [plsc.load_gather(
                        xb, [jnp.full((LN,), j * _C + t, jnp.int32)])
                     for j in range(4)]
                for cc in range(D // LN):
                    v = bvec[cc]
                    for j in range(4):
                        v = v + n[j] * wvec[cc][j]
                    acc[t, pl.ds(cc * LN, LN)] = v
                return carry

            lax.fori_loop(0, _C, tokbody, 0)

        def fire(s):
            idx_v, acc = idxs[s], accs[s]
            return [pltpu.async_copy(
                tables[i].at[idx_v.at[i]], acc, gsem, add=True)
                for i in range(4)]

        def body(g2, carry):
            l_a = g2 * 2
            d_a = fire(0)

            @pl.when(g2 > 0)
            def _():
                scatter_desc(1).wait()

            prepare(1, l_a + 1, drain=False)
            for cp in d_a:
                cp.wait()
            d_b = fire(1)
            sc_a = scatter_desc(0)
            sc_a.start()

            @pl.when(g2 < n2 - 1)
            def _():
                prepare(0, l_a + 2, drain=True)

            for cp in d_b:
                cp.wait()
            sc_b = scatter_desc(1)
            sc_b.start()
            return carry

        prepare(0, 0, drain=False)
        lax.fori_loop(0, n2, body, 0, unroll=False)
        scatter_desc(0).wait()
        scatter_desc(1).wait()

    return k(p1, p2, p3, p4, xt1, w0b)


def kernel(x, emb_region, emb_gene, emb_strand, emb_exon, W_proj, b_proj):
    xt1 = jnp.transpose(x, (2, 1, 0)).reshape(NF * NL * NB)
    w1 = W_proj[4:132]
    w2 = W_proj[132:260]
    w3 = W_proj[260:388]
    w4 = W_proj[388:516]
    w0b = jnp.concatenate([W_proj[:4], b_proj.reshape(1, D)], axis=0)
    p1, p2, p3, p4 = _project_tables(
        emb_region, emb_gene, emb_strand, emb_exon, w1, w2, w3, w4)
    out = _gather_sum(p1, p2, p3, p4, xt1, w0b)
    return out.reshape(NB, NL, D)


# speedup vs baseline: 20.9572x; 1.0022x over previous
import functools

import jax
import jax.numpy as jnp
from jax import lax
from jax.experimental import pallas as pl
from jax.experimental.pallas import tpu as pltpu
from jax.experimental.pallas import tpu_sc as plsc

NUM_EMB = 100000
D = 128
NB = 4096
NL = 200
T = NB * NL
NF = 9
LN = 16


_ROWS_BLK = 4000
_PROJ_STEPS = NUM_EMB // _ROWS_BLK


def _proj_body(e1, e2, e3, e4, w1, w2, w3, w4, p1, p2, p3, p4):
    p1[...] = jnp.dot(e1[...], w1[...], preferred_element_type=jnp.float32)
    p2[...] = jnp.dot(e2[...], w2[...], preferred_element_type=jnp.float32)
    p3[...] = jnp.dot(e3[...], w3[...], preferred_element_type=jnp.float32)
    p4[...] = jnp.dot(e4[...], w4[...], preferred_element_type=jnp.float32)


def _project_tables(e1, e2, e3, e4, w1, w2, w3, w4):
    blk = pl.BlockSpec((_ROWS_BLK, D), lambda i: (i, 0))
    wblk = pl.BlockSpec((D, D), lambda i: (0, 0))
    shp = jax.ShapeDtypeStruct((NUM_EMB, D), jnp.float32)
    return pl.pallas_call(
        _proj_body,
        grid=(_PROJ_STEPS,),
        in_specs=[blk, blk, blk, blk, wblk, wblk, wblk, wblk],
        out_specs=[blk, blk, blk, blk],
        out_shape=[shp, shp, shp, shp],
    )(e1, e2, e3, e4, w1, w2, w3, w4)


_C = 128


def _gather_sum(p1, p2, p3, p4, xt1, w0b):
    info = plsc.get_sparse_core_info()
    nc, ns = info.num_cores, info.num_subcores
    nw = nc * ns
    assert NB == nw * _C
    n2 = NL // 2
    mesh = plsc.VectorSubcoreMesh(core_axis_name="c", subcore_axis_name="s")

    @functools.partial(
        pl.kernel,
        out_type=jax.ShapeDtypeStruct((T, D), jnp.float32),
        mesh=mesh,
        scratch_types=[
            pltpu.VMEM((8 * _C,), jnp.float32),
            pltpu.VMEM((8 * _C,), jnp.float32),
            pltpu.VMEM((5, _C), jnp.int32),
            pltpu.VMEM((5, _C), jnp.int32),
            pltpu.VMEM((_C, D), jnp.float32),
            pltpu.VMEM((_C, D), jnp.float32),
            pltpu.VMEM((5, D), jnp.float32),
            pltpu.SemaphoreType.DMA,
            pltpu.SemaphoreType.DMA,
            pltpu.SemaphoreType.DMA,
            pltpu.SemaphoreType.DMA,
            pltpu.SemaphoreType.DMA,
        ],
        compiler_params=pltpu.CompilerParams(needs_layout_passes=False),
    )
    def k(p1h, p2h, p3h, p4h, xh, wh, outh,
          xb0, xb1, idx0, idx1, acc0, acc1, w0v,
          xsem, gsem0, gsem1, osem0, osem1):
        gsems = (gsem0, gsem1)
        wid = lax.axis_index("s") * nc + lax.axis_index("c")
        b0 = wid * _C
        tables = (p1h, p2h, p3h, p4h)
        xbs, idxs, accs = (xb0, xb1), (idx0, idx1), (acc0, acc1)
        osems = (osem0, osem1)

        pltpu.sync_copy(wh, w0v)
        iota200 = lax.iota(jnp.int32, LN) * NL
        wvec = [[w0v[j, pl.ds(cc * LN, LN)] for j in range(4)]
                for cc in range(D // LN)]
        bvec = [w0v[4, pl.ds(cc * LN, LN)] for cc in range(D // LN)]

        def scatter_desc(s):
            return pltpu.make_async_copy(
                accs[s], outh.at[idxs[s].at[4]], osems[s])

        def prepare(s, l, drain):
            xb, idx_v, acc = xbs[s], idxs[s], accs[s]
            dx = []
            for j in range(8):
                off = pl.multiple_of((j * NL + l) * NB + b0, _C)
                dx.append(pltpu.async_copy(
                    xh.at[pl.ds(off, _C)], xb.at[pl.ds(j * _C, _C)], xsem))
            for d in dx:
                d.wait()
            for kk in range(4):
                for m in range(_C // LN):
                    v = xb[pl.ds((4 + kk) * _C + m * LN, LN)]
                    idx_v[kk, pl.ds(m * LN, LN)] = v.astype(jnp.int32)
            for m in range(_C // LN):
                idx_v[4, pl.ds(m * LN, LN)] = (
                    iota200 + ((b0 + m * LN) * NL + l))
            if drain:
                scatter_desc(s).wait()
            def tokbody(t, carry):
                n = [plsc.load_gather(
                        xb, [jnp.full((LN,), j * _C + t, jnp.int32)])
                     for j in range(4)]
                for cc in range(D // LN):
                    v = bvec[cc]
                    for j in range(4):
                        v = v + n[j] * wvec[cc][j]
                    acc[t, pl.ds(cc * LN, LN)] = v
                return carry

            lax.fori_loop(0, _C, tokbody, 0)

        def fire(s):
            idx_v, acc = idxs[s], accs[s]
            return [pltpu.async_copy(
                tables[i].at[idx_v.at[i]], acc, gsems[s], add=True)
                for i in range(4)]

        def body(g2, carry):
            l_a = g2 * 2
            d_a = fire(0)

            @pl.when(g2 > 0)
            def _():
                scatter_desc(1).wait()

            prepare(1, l_a + 1, drain=False)
            d_b = fire(1)
            for cp in d_a:
                cp.wait()
            sc_a = scatter_desc(0)
            sc_a.start()

            @pl.when(g2 < n2 - 1)
            def _():
                prepare(0, l_a + 2, drain=True)

            for cp in d_b:
                cp.wait()
            sc_b = scatter_desc(1)
            sc_b.start()
            return carry

        prepare(0, 0, drain=False)
        lax.fori_loop(0, n2, body, 0, unroll=False)
        scatter_desc(0).wait()
        scatter_desc(1).wait()

    return k(p1, p2, p3, p4, xt1, w0b)


def kernel(x, emb_region, emb_gene, emb_strand, emb_exon, W_proj, b_proj):
    xt1 = jnp.transpose(x, (2, 1, 0)).reshape(NF * NL * NB)
    w1 = W_proj[4:132]
    w2 = W_proj[132:260]
    w3 = W_proj[260:388]
    w4 = W_proj[388:516]
    w0b = jnp.concatenate([W_proj[:4], b_proj.reshape(1, D)], axis=0)
    p1, p2, p3, p4 = _project_tables(
        emb_region, emb_gene, emb_strand, emb_exon, w1, w2, w3, w4)
    out = _gather_sum(p1, p2, p3, p4, xt1, w0b)
    return out.reshape(NB, NL, D)
